# Initial kernel scaffold; baseline (speedup 1.0000x reference)
import jax
import jax.numpy as jnp
from jax.experimental import pallas as pl


def kernel(x, s, edge_index, gW0, gb0, gW1, gb1, gW2, gb2, n1_w1, n1_b1, n1_w2, n1_b2, n1_a, n2_w1, n2_b1, n2_w2, n2_b2, n2_a, n3_w1, n3_b1, n3_w2, n3_b2, n3_a):
    raise NotImplementedError("write your pallas kernel here")


# speedup vs baseline: 24.3865x; 24.3865x over previous
import functools

import jax
import jax.numpy as jnp
from jax import lax
from jax.experimental import pallas as pl
from jax.experimental.pallas import tpu as pltpu
from jax.experimental.pallas import tpu_sc as plsc

N = 10000
E = 320000
A_DIM = 128
S_DIM = 64
A_HID = 64
S_HID = 32

NW = 32
K = 128
ETOT = E + N
S_STEPS = -(-ETOT // (NW * K))
EPAD = NW * K * S_STEPS
NACC = 10112
PAD_ROW = N + 1
ROWS_PER_SUB = NACC // 16


def _make_segsum(D):
  mesh = plsc.VectorSubcoreMesh(core_axis_name="c", subcore_axis_name="s")

  @functools.partial(
      pl.kernel,
      mesh=mesh,
      out_type=jax.ShapeDtypeStruct((2, NACC, D), jnp.float32),
      compiler_params=pltpu.CompilerParams(use_tc_tiling_on_sc=False),
      scratch_types=[
          pltpu.VMEM((S_STEPS, K), jnp.int32),
          pltpu.VMEM((S_STEPS, K), jnp.int32),
          pltpu.VMEM((K, D), jnp.float32),
          pltpu.VMEM_SHARED((NACC, D), jnp.float32),
          pltpu.SemaphoreType.DMA,
      ],
  )
  def seg(table_hbm, src_hbm, dst_hbm, zeros_hbm, out_hbm,
          src_v, dst_v, rows_v, acc_sh, sem):
    c = lax.axis_index("c")
    s = lax.axis_index("s")
    wid = c * 16 + s
    pltpu.sync_copy(zeros_hbm.at[pl.ds(s * ROWS_PER_SUB, ROWS_PER_SUB)],
                    acc_sh.at[pl.ds(s * ROWS_PER_SUB, ROWS_PER_SUB)])
    pltpu.sync_copy(src_hbm.at[wid], src_v)
    pltpu.sync_copy(dst_hbm.at[wid], dst_v)
    plsc.subcore_barrier()

    def body(j, carry):
      pltpu.async_copy(table_hbm.at[src_v.at[j]], rows_v, sem).wait()
      pltpu.sync_copy(rows_v, acc_sh.at[dst_v.at[j]], add=True)
      return carry

    lax.fori_loop(0, S_STEPS, body, 0)
    plsc.subcore_barrier()
    pltpu.sync_copy(acc_sh.at[pl.ds(s * ROWS_PER_SUB, ROWS_PER_SUB)],
                    out_hbm.at[c].at[pl.ds(s * ROWS_PER_SUB, ROWS_PER_SUB)])

  return seg


_SEGSUM = {}


def _segsum(table, src3, dst3):
  d = table.shape[1]
  if d not in _SEGSUM:
    _SEGSUM[d] = _make_segsum(d)
  zeros = jnp.zeros((NACC, d), jnp.float32)
  return _SEGSUM[d](table, src3, dst3, zeros)


def _tc(fn, out_shapes, *ins):
  return pl.pallas_call(fn, out_shape=out_shapes)(*ins)


def _k_dinv(acc_ref, o_ref):
  deg = acc_ref[0, :N, :1] + acc_ref[1, :N, :1]
  o_ref[...] = 1.0 / jnp.sqrt(deg)


def _k_gcn_pre(x_ref, w_ref, dinv_ref, o_ref):
  h = jnp.dot(x_ref[...], w_ref[...], preferred_element_type=jnp.float32)
  o_ref[...] = h * dinv_ref[...]


def _mk_gcn_post(relu):
  def _k(acc_ref, dinv_ref, b_ref, o_ref):
    t = acc_ref[0, :N, :] + acc_ref[1, :N, :]
    o = t * dinv_ref[...] + b_ref[...]
    o_ref[...] = jnp.maximum(o, 0.0) if relu else o
  return _k


def _mk_gna_pre(do, dp):
  def _k(z_ref, w1_ref, b1_ref, w2_ref, b2_ref, a_ref, lin_ref, tab_ref):
    z = z_ref[...]
    h = jnp.dot(z, w2_ref[...], preferred_element_type=jnp.float32) + b2_ref[...]
    p = jnp.dot(h, a_ref[...], preferred_element_type=jnp.float32)
    bw = jnp.exp(jnp.min(p) - p)
    lin_ref[...] = jnp.dot(z, w1_ref[...],
                           preferred_element_type=jnp.float32) + b1_ref[...]
    pad = jnp.zeros((N, dp - do - 1), jnp.float32)
    tab_ref[...] = jnp.concatenate([h * bw, bw, pad], axis=1)
  return _k


def _mk_gna_post(do):
  def _k(acc_ref, lin_ref, o_ref):
    t = acc_ref[0, :N, :] + acc_ref[1, :N, :]
    numer = t[:, :do]
    den = t[:, do:do + 1]
    o_ref[...] = jnp.maximum(lin_ref[...] + numer / den, 0.0)
  return _k


def _gcn_layer(x, w, b, dinv, src3, dst3, relu):
  t = _tc(_k_gcn_pre, jax.ShapeDtypeStruct((N, w.shape[1]), jnp.float32),
          x, w, dinv)
  acc = _segsum(t, src3, dst3)
  return _tc(_mk_gcn_post(relu),
             jax.ShapeDtypeStruct((N, w.shape[1]), jnp.float32),
             acc, dinv, b)


def _gna_layer(z, w1, b1, w2, b2, a, src3, dst3):
  do = w1.shape[1]
  dp = {32: 48, 64: 80}[do]
  lin, tab = _tc(_mk_gna_pre(do, dp),
                 (jax.ShapeDtypeStruct((N, do), jnp.float32),
                  jax.ShapeDtypeStruct((N, dp), jnp.float32)),
                 z, w1, b1, w2, b2, a)
  acc = _segsum(tab, src3, dst3)
  return _tc(_mk_gna_post(do), jax.ShapeDtypeStruct((N, do), jnp.float32),
             acc, lin)


def kernel(x, s, edge_index, gW0, gb0, gW1, gb1, gW2, gb2,
           n1_w1, n1_b1, n1_w2, n1_b2, n1_a,
           n2_w1, n2_b1, n2_w2, n2_b2, n2_a,
           n3_w1, n3_b1, n3_w2, n3_b2, n3_a):
  loop = jnp.arange(N, dtype=jnp.int32)
  pad = EPAD - ETOT
  src = jnp.concatenate(
      [edge_index[0], loop, jnp.zeros((pad,), jnp.int32)])
  dst = jnp.concatenate(
      [edge_index[1], loop, jnp.full((pad,), PAD_ROW, jnp.int32)])
  src3 = src.reshape(NW, S_STEPS, K)
  dst3 = dst.reshape(NW, S_STEPS, K)

  ones_tab = jnp.ones((N, 16), jnp.float32)
  deg_acc = _segsum(ones_tab, src3, dst3)
  dinv = _tc(_k_dinv, jax.ShapeDtypeStruct((N, 1), jnp.float32), deg_acc)

  h = _gcn_layer(x, gW0, gb0, dinv, src3, dst3, relu=True)
  h = _gcn_layer(h, gW1, gb1, dinv, src3, dst3, relu=True)
  x_ = _gcn_layer(h, gW2, gb2, dinv, src3, dst3, relu=False)

  z = _gna_layer(s, n1_w1, n1_b1, n1_w2, n1_b2, n1_a, src3, dst3)
  z = _gna_layer(z, n2_w1, n2_b1, n2_w2, n2_b2, n2_a, src3, dst3)
  s_ = _gna_layer(z, n3_w1, n3_b1, n3_w2, n3_b2, n3_a, src3, dst3)
  return (x_, s_)
